# baseline (speedup 1.0000x reference)
import functools

import jax
import jax.numpy as jnp
from jax import lax
from jax.experimental import pallas as pl
from jax.experimental.pallas import tpu as pltpu
from jax.experimental.pallas import tpu_sc as plsc

NENTITY = 1000000
NRELATION = 1000
DIM = 64
BATCH = 1024
NEG = 256

L = 16
NC = 2
NS = 16
NW = NC * NS
BPW = BATCH // NW
NEG_HALF = NEG // 2
DIM2 = 2 * DIM

_mesh = plsc.VectorSubcoreMesh(core_axis_name="c", subcore_axis_name="s")


@functools.partial(
    pl.kernel,
    mesh=_mesh,
    compiler_params=pltpu.CompilerParams(use_tc_tiling_on_sc=False),
    out_type=jax.ShapeDtypeStruct((BATCH, NEG), jnp.float32),
    scratch_types=[
        pltpu.VMEM((BPW,), jnp.int32),
        pltpu.VMEM((BPW,), jnp.int32),
        pltpu.VMEM((BPW,), jnp.int32),
        pltpu.VMEM((BPW, 2, NEG_HALF), jnp.int32),
        pltpu.VMEM((BPW, 2, NEG_HALF), jnp.int32),
        pltpu.VMEM((BPW, DIM2), jnp.float32),
        pltpu.VMEM((BPW, DIM2), jnp.float32),
        pltpu.VMEM((BPW, DIM), jnp.float32),
        pltpu.VMEM((NEG, DIM2), jnp.float32),
        pltpu.VMEM((NEG, DIM2), jnp.float32),
        pltpu.VMEM((BPW, NEG), jnp.float32),
        pltpu.SemaphoreType.DMA,
        pltpu.SemaphoreType.DMA,
    ],
)
def _kge_sc(hidx_hbm, ridx_hbm, neg_hbm, ent_hbm, rel_hbm, out_hbm,
            hidx_v, hgix_v, ridx_v, nidx_v, gidx_v, hrows_v, rrows_v,
            hr_v, tail0_v, tail1_v, score_v, sem0, sem1):
    wid = lax.axis_index("s") * NC + lax.axis_index("c")
    base = wid * BPW

    pltpu.sync_copy(hidx_hbm.at[pl.ds(base, BPW)], hidx_v)
    pltpu.sync_copy(ridx_hbm.at[pl.ds(base, BPW)], ridx_v)
    pltpu.sync_copy(neg_hbm.at[pl.ds(base, BPW)], nidx_v)

    for k in range(BPW // L):
        sl = pl.ds(k * L, L)
        hgix_v[sl] = lax.shift_right_logical(hidx_v[sl], 1)

    def shift_body(b, carry):
        for h in range(2):
            for k in range(NEG_HALF // L):
                sl = pl.ds(k * L, L)
                gidx_v[b, h, sl] = lax.shift_right_logical(nidx_v[b, h, sl], 1)
        return carry

    lax.fori_loop(0, BPW, shift_body, 0)

    pltpu.async_copy(ent_hbm.at[hgix_v], hrows_v, sem0).wait()
    pltpu.async_copy(rel_hbm.at[ridx_v], rrows_v, sem0).wait()

    lanes = lax.iota(jnp.int32, L)
    dnums = lax.GatherDimensionNumbers(
        offset_dims=(), collapsed_slice_dims=(0,), start_index_map=(0,))
    perm_idx = {k: lanes ^ k for k in (1, 2, 4, 8)}
    merge_mask = {k: (lanes & k) != 0 for k in (1, 2, 4, 8)}

    def merge(a, b, k):
        m = merge_mask[k]
        t = jnp.where(m, b, a)
        u = jnp.where(m, a, b)
        return t + lax.gather(u, perm_idx[k][:, None], dnums, (1,),
                              mode=lax.GatherScatterMode.PROMISE_IN_BOUNDS)

    def tree(s):
        t = [merge(s[2 * i], s[2 * i + 1], 1) for i in range(8)]
        u = [merge(t[2 * i], t[2 * i + 1], 2) for i in range(4)]
        v = [merge(u[2 * i], u[2 * i + 1], 4) for i in range(2)]
        return merge(v[0], v[1], 8)

    for b in range(BPW):
        hv = hidx_v[pl.ds((b // L) * L, L)]
        pb = lax.gather(hv, jnp.full((L, 1), b % L, jnp.int32), dnums, (1,),
                        mode=lax.GatherScatterMode.PROMISE_IN_BOUNDS)
        pf = lax.convert_element_type(lax.bitwise_and(pb, 1), jnp.float32)
        for k in range(DIM // L):
            lo = hrows_v[b, pl.ds(k * L, L)]
            hi = hrows_v[b, pl.ds(DIM + k * L, L)]
            hr_v[b, pl.ds(k * L, L)] = (
                (lo + pf * (hi - lo)) * rrows_v[b, pl.ds(k * L, L)])

    def tail_copies(b, buf, sem):
        return (
            pltpu.make_async_copy(ent_hbm.at[gidx_v.at[b, 0]],
                                  buf.at[pl.ds(0, NEG_HALF)], sem),
            pltpu.make_async_copy(ent_hbm.at[gidx_v.at[b, 1]],
                                  buf.at[pl.ds(NEG_HALF, NEG_HALF)], sem),
        )

    def start_tails(b, buf, sem):
        for cp in tail_copies(b, buf, sem):
            cp.start()

    def wait_tails(b, buf, sem):
        for cp in tail_copies(b, buf, sem):
            cp.wait()

    def compute_row(b, tail_v):
        hr = [hr_v[b, pl.ds(k * L, L)] for k in range(DIM // L)]

        def grp_body(g, gcarry):
            n0 = g * L
            par = nidx_v[b, lax.shift_right_logical(g, 3),
                         pl.ds(lax.mul(lax.rem(g, 8), L), L)]
            pf = lax.convert_element_type(lax.bitwise_and(par, 1),
                                          jnp.float32)
            slo, shi = [], []
            for j in range(L):
                n = n0 + j
                slo.append(tail_v[n, pl.ds(0 * L, L)] * hr[0]
                           + tail_v[n, pl.ds(1 * L, L)] * hr[1]
                           + tail_v[n, pl.ds(2 * L, L)] * hr[2]
                           + tail_v[n, pl.ds(3 * L, L)] * hr[3])
                shi.append(tail_v[n, pl.ds(4 * L, L)] * hr[0]
                           + tail_v[n, pl.ds(5 * L, L)] * hr[1]
                           + tail_v[n, pl.ds(6 * L, L)] * hr[2]
                           + tail_v[n, pl.ds(7 * L, L)] * hr[3])
            tlo = tree(slo)
            score_v[b, pl.ds(n0, L)] = tlo + pf * (tree(shi) - tlo)
            return gcarry

        lax.fori_loop(0, NEG // L, grp_body, 0)

    start_tails(0, tail0_v, sem0)

    def pair_body(i, carry):
        b0 = 2 * i
        b1 = b0 + 1
        start_tails(b1, tail1_v, sem1)
        wait_tails(b0, tail0_v, sem0)
        compute_row(b0, tail0_v)

        @pl.when(i < BPW // 2 - 1)
        def _():
            start_tails(b0 + 2, tail0_v, sem0)

        wait_tails(b1, tail1_v, sem1)
        compute_row(b1, tail1_v)
        return carry

    lax.fori_loop(0, BPW // 2, pair_body, 0)
    pltpu.sync_copy(score_v, out_hbm.at[pl.ds(base, BPW)])


def kernel(pos_part, neg_part, entity_embedding, relation_embedding):
    hidx = pos_part[:, 0].astype(jnp.int32)
    ridx = pos_part[:, 1].astype(jnp.int32)
    neg3 = neg_part.astype(jnp.int32).reshape(BATCH, 2, NEG_HALF)
    ent2 = entity_embedding.reshape(NENTITY // 2, DIM2)
    rel2 = jnp.concatenate([relation_embedding, relation_embedding], axis=1)
    return _kge_sc(hidx, ridx, neg3, ent2, rel2)


# speedup vs baseline: 1.2085x; 1.2085x over previous
import functools

import jax
import jax.numpy as jnp
from jax import lax
from jax.experimental import pallas as pl
from jax.experimental.pallas import tpu as pltpu
from jax.experimental.pallas import tpu_sc as plsc

NENTITY = 1000000
NRELATION = 1000
DIM = 64
BATCH = 1024
NEG = 256

L = 16
NC = 2
NS = 16
NW = NC * NS
BPW = BATCH // NW
NEG_HALF = NEG // 2

_mesh = plsc.VectorSubcoreMesh(core_axis_name="c", subcore_axis_name="s")


@functools.partial(
    pl.kernel,
    mesh=_mesh,
    compiler_params=pltpu.CompilerParams(use_tc_tiling_on_sc=False),
    out_type=jax.ShapeDtypeStruct((BATCH, NEG), jnp.float32),
    scratch_types=[
        pltpu.VMEM((BPW,), jnp.int32),
        pltpu.VMEM((BPW,), jnp.int32),
        pltpu.VMEM((BPW, 2, NEG_HALF), jnp.int32),
        pltpu.VMEM((BPW, DIM), jnp.float32),
        pltpu.VMEM((BPW, DIM), jnp.float32),
        pltpu.VMEM((NEG, DIM), jnp.float32),
        pltpu.VMEM((NEG, DIM), jnp.float32),
        pltpu.VMEM((BPW, NEG), jnp.float32),
        pltpu.SemaphoreType.DMA,
        pltpu.SemaphoreType.DMA,
    ],
)
def _kge_sc(hidx_hbm, ridx_hbm, neg_hbm, ent_hbm, rel_hbm, out_hbm,
            hidx_v, ridx_v, nidx_v, hrows_v, rrows_v, tail0_v, tail1_v,
            score_v, sem0, sem1):
    wid = lax.axis_index("s") * NC + lax.axis_index("c")
    base = wid * BPW

    pltpu.sync_copy(hidx_hbm.at[pl.ds(base, BPW)], hidx_v)
    pltpu.sync_copy(ridx_hbm.at[pl.ds(base, BPW)], ridx_v)
    pltpu.sync_copy(neg_hbm.at[pl.ds(base, BPW)], nidx_v)
    pltpu.async_copy(ent_hbm.at[hidx_v], hrows_v, sem0).wait()
    pltpu.async_copy(rel_hbm.at[ridx_v], rrows_v, sem0).wait()

    def hr_body(b, carry):
        for k in range(DIM // L):
            sl = pl.ds(k * L, L)
            rrows_v[b, sl] = hrows_v[b, sl] * rrows_v[b, sl]
        return carry

    lax.fori_loop(0, BPW, hr_body, 0)

    def tail_copies(b, buf, sem):
        return (
            pltpu.make_async_copy(ent_hbm.at[nidx_v.at[b, 0]],
                                  buf.at[pl.ds(0, NEG_HALF)], sem),
            pltpu.make_async_copy(ent_hbm.at[nidx_v.at[b, 1]],
                                  buf.at[pl.ds(NEG_HALF, NEG_HALF)], sem),
        )

    def start_tails(b, buf, sem):
        for cp in tail_copies(b, buf, sem):
            cp.start()

    def wait_tails(b, buf, sem):
        for cp in tail_copies(b, buf, sem):
            cp.wait()

    lanes = lax.iota(jnp.int32, L)
    dnums = lax.GatherDimensionNumbers(
        offset_dims=(), collapsed_slice_dims=(0,), start_index_map=(0,))
    perm_idx = {k: lanes ^ k for k in (1, 2, 4, 8)}
    merge_mask = {k: (lanes & k) != 0 for k in (1, 2, 4, 8)}

    def merge(a, b, k):
        m = merge_mask[k]
        t = jnp.where(m, b, a)
        u = jnp.where(m, a, b)
        return t + lax.gather(u, perm_idx[k][:, None], dnums, (1,),
                              mode=lax.GatherScatterMode.PROMISE_IN_BOUNDS)

    def compute_row(b, tail_v):
        hr0 = rrows_v[b, pl.ds(0 * L, L)]
        hr1 = rrows_v[b, pl.ds(1 * L, L)]
        hr2 = rrows_v[b, pl.ds(2 * L, L)]
        hr3 = rrows_v[b, pl.ds(3 * L, L)]

        def grp_body(g, gcarry):
            n0 = g * L
            s = []
            for j in range(L):
                n = n0 + j
                s.append(tail_v[n, pl.ds(0 * L, L)] * hr0
                         + tail_v[n, pl.ds(1 * L, L)] * hr1
                         + tail_v[n, pl.ds(2 * L, L)] * hr2
                         + tail_v[n, pl.ds(3 * L, L)] * hr3)
            t = [merge(s[2 * i], s[2 * i + 1], 1) for i in range(8)]
            u = [merge(t[2 * i], t[2 * i + 1], 2) for i in range(4)]
            v = [merge(u[2 * i], u[2 * i + 1], 4) for i in range(2)]
            score_v[b, pl.ds(n0, L)] = merge(v[0], v[1], 8)
            return gcarry

        lax.fori_loop(0, NEG // L, grp_body, 0)

    start_tails(0, tail0_v, sem0)

    def pair_body(i, carry):
        b0 = 2 * i
        b1 = b0 + 1
        start_tails(b1, tail1_v, sem1)
        wait_tails(b0, tail0_v, sem0)
        compute_row(b0, tail0_v)

        @pl.when(i < BPW // 2 - 1)
        def _():
            start_tails(b0 + 2, tail0_v, sem0)

        wait_tails(b1, tail1_v, sem1)
        compute_row(b1, tail1_v)
        return carry

    lax.fori_loop(0, BPW // 2, pair_body, 0)
    pltpu.sync_copy(score_v, out_hbm.at[pl.ds(base, BPW)])


CB = 128
NBLK_T = NENTITY // CB
NPAIR_T = 122
NEXTRA_T = NBLK_T - 2 * NPAIR_T * NW
NREM = NENTITY - NBLK_T * CB


@functools.partial(
    pl.kernel,
    mesh=_mesh,
    compiler_params=pltpu.CompilerParams(needs_layout_passes=False),
    out_type=jax.ShapeDtypeStruct((NENTITY // 2, 2 * DIM), jnp.float32),
    scratch_types=[
        pltpu.VMEM((DIM, CB), jnp.float32),
        pltpu.VMEM((DIM, CB), jnp.float32),
        pltpu.VMEM((DIM, NREM), jnp.float32),
        pltpu.VMEM((DIM, CB), jnp.float32),
        pltpu.VMEM((DIM, CB), jnp.float32),
        pltpu.SemaphoreType.DMA,
        pltpu.SemaphoreType.DMA,
        pltpu.SemaphoreType.DMA,
        pltpu.SemaphoreType.DMA,
    ],
)
def _relayout(entt_hbm, out_hbm, in0, in1, inr, out0, out1,
              rd0, rd1, wr0, wr1):
    wid = lax.axis_index("s") * NC + lax.axis_index("c")
    lanes = lax.iota(jnp.int32, L)
    cc = [g * L + lanes for g in range(CB // L)]
    cst = [(g * L + lanes) * DIM for g in range(CB // L)]

    def read_cp(blk, buf, sem):
        return pltpu.make_async_copy(
            entt_hbm.at[:, pl.ds(blk * CB, CB)], buf, sem)

    def write_cp(blk, buf, sem):
        return pltpu.make_async_copy(
            buf, out_hbm.at[pl.ds(blk * (CB // 2), CB // 2)], sem)

    def transpose_block(inb, outb, ngrp):
        def kbody(k, dv):
            for g in range(ngrp):
                v = plsc.load_gather(inb, [dv, cc[g]])
                flat = cst[g] + dv
                plsc.store_scatter(
                    outb,
                    [lax.shift_right_logical(flat, 7),
                     lax.bitwise_and(flat, CB - 1)],
                    v)
            return lax.bitwise_and(dv + 1, DIM - 1)
        lax.fori_loop(0, DIM, kbody, lanes)

    read_cp(wid, in0, rd0).start()
    read_cp(wid + NW, in1, rd1).start()

    def pair_body(i, carry):
        j0 = wid + (2 * i) * NW
        j1 = j0 + NW
        read_cp(j0, in0, rd0).wait()

        @pl.when(i > 0)
        def _():
            write_cp(j0 - 2 * NW, out0, wr0).wait()

        transpose_block(in0, out0, CB // L)

        @pl.when(i < NPAIR_T - 1)
        def _():
            read_cp(j0 + 2 * NW, in0, rd0).start()

        write_cp(j0, out0, wr0).start()
        read_cp(j1, in1, rd1).wait()

        @pl.when(i > 0)
        def _():
            write_cp(j1 - 2 * NW, out1, wr1).wait()

        transpose_block(in1, out1, CB // L)

        @pl.when(i < NPAIR_T - 1)
        def _():
            read_cp(j1 + 2 * NW, in1, rd1).start()

        write_cp(j1, out1, wr1).start()
        return carry

    lax.fori_loop(0, NPAIR_T, pair_body, 0)
    last0 = wid + (2 * NPAIR_T - 2) * NW
    write_cp(last0, out0, wr0).wait()
    write_cp(last0 + NW, out1, wr1).wait()

    @pl.when(wid < NEXTRA_T)
    def _():
        jx = 2 * NPAIR_T * NW + wid
        read_cp(jx, in0, rd0).start()
        read_cp(jx, in0, rd0).wait()
        transpose_block(in0, out0, CB // L)
        write_cp(jx, out0, wr0).start()
        write_cp(jx, out0, wr0).wait()

    @pl.when(wid == NW - 1)
    def _():
        c0 = NBLK_T * CB
        rcp = pltpu.make_async_copy(
            entt_hbm.at[:, pl.ds(c0, NREM)], inr, rd1)
        rcp.start()
        rcp.wait()
        transpose_block(inr, out1, NREM // L)
        wcp = pltpu.make_async_copy(
            out1.at[pl.ds(0, NREM // 2)],
            out_hbm.at[pl.ds(c0 // 2, NREM // 2)], wr1)
        wcp.start()
        wcp.wait()


def kernel(pos_part, neg_part, entity_embedding, relation_embedding):
    hidx = pos_part[:, 0].astype(jnp.int32)
    ridx = pos_part[:, 1].astype(jnp.int32)
    neg3 = neg_part.astype(jnp.int32).reshape(BATCH, 2, NEG_HALF)
    ent_lin = _relayout(entity_embedding.T).reshape(NENTITY, DIM)
    return _kge_sc(hidx, ridx, neg3, ent_lin, relation_embedding)


# speedup vs baseline: 2.8100x; 2.3252x over previous
import functools

import jax
import jax.numpy as jnp
from jax import lax
from jax.experimental import pallas as pl
from jax.experimental.pallas import tpu as pltpu
from jax.experimental.pallas import tpu_sc as plsc

NENTITY = 1000000
NRELATION = 1000
DIM = 64
BATCH = 1024
NEG = 256

L = 16
NC = 2
NS = 16
NW = NC * NS
BPW = BATCH // NW
NEG_HALF = NEG // 2

_mesh = plsc.VectorSubcoreMesh(core_axis_name="c", subcore_axis_name="s")


@functools.partial(
    pl.kernel,
    mesh=_mesh,
    compiler_params=pltpu.CompilerParams(use_tc_tiling_on_sc=False),
    out_type=jax.ShapeDtypeStruct((BATCH, NEG), jnp.float32),
    scratch_types=[
        pltpu.VMEM((BPW,), jnp.int32),
        pltpu.VMEM((BPW,), jnp.int32),
        pltpu.VMEM((BPW, 2, NEG_HALF), jnp.int32),
        pltpu.VMEM((BPW, DIM), jnp.float32),
        pltpu.VMEM((BPW, DIM), jnp.float32),
        pltpu.VMEM((NEG, DIM), jnp.float32),
        pltpu.VMEM((NEG, DIM), jnp.float32),
        pltpu.VMEM((BPW, NEG), jnp.float32),
        pltpu.SemaphoreType.DMA,
        pltpu.SemaphoreType.DMA,
    ],
)
def _kge_sc(hidx_hbm, ridx_hbm, neg_hbm, ent_hbm, rel_hbm, out_hbm,
            hidx_v, ridx_v, nidx_v, hrows_v, rrows_v, tail0_v, tail1_v,
            score_v, sem0, sem1):
    wid = lax.axis_index("s") * NC + lax.axis_index("c")
    base = wid * BPW

    pltpu.sync_copy(hidx_hbm.at[pl.ds(base, BPW)], hidx_v)
    pltpu.sync_copy(ridx_hbm.at[pl.ds(base, BPW)], ridx_v)
    pltpu.sync_copy(neg_hbm.at[pl.ds(base, BPW)], nidx_v)
    pltpu.async_copy(ent_hbm.at[hidx_v], hrows_v, sem0).wait()
    pltpu.async_copy(rel_hbm.at[ridx_v], rrows_v, sem0).wait()

    def hr_body(b, carry):
        for k in range(DIM // L):
            sl = pl.ds(k * L, L)
            rrows_v[b, sl] = hrows_v[b, sl] * rrows_v[b, sl]
        return carry

    lax.fori_loop(0, BPW, hr_body, 0)

    def tail_copies(b, buf, sem):
        return (
            pltpu.make_async_copy(ent_hbm.at[nidx_v.at[b, 0]],
                                  buf.at[pl.ds(0, NEG_HALF)], sem),
            pltpu.make_async_copy(ent_hbm.at[nidx_v.at[b, 1]],
                                  buf.at[pl.ds(NEG_HALF, NEG_HALF)], sem),
        )

    def start_tails(b, buf, sem):
        for cp in tail_copies(b, buf, sem):
            cp.start()

    def wait_tails(b, buf, sem):
        for cp in tail_copies(b, buf, sem):
            cp.wait()

    lanes = lax.iota(jnp.int32, L)
    dnums = lax.GatherDimensionNumbers(
        offset_dims=(), collapsed_slice_dims=(0,), start_index_map=(0,))
    perm_idx = {k: lanes ^ k for k in (1, 2, 4, 8)}
    merge_mask = {k: (lanes & k) != 0 for k in (1, 2, 4, 8)}

    def merge(a, b, k):
        m = merge_mask[k]
        t = jnp.where(m, b, a)
        u = jnp.where(m, a, b)
        return t + lax.gather(u, perm_idx[k][:, None], dnums, (1,),
                              mode=lax.GatherScatterMode.PROMISE_IN_BOUNDS)

    def compute_row(b, tail_v):
        hr0 = rrows_v[b, pl.ds(0 * L, L)]
        hr1 = rrows_v[b, pl.ds(1 * L, L)]
        hr2 = rrows_v[b, pl.ds(2 * L, L)]
        hr3 = rrows_v[b, pl.ds(3 * L, L)]

        @plsc.parallel_loop(0, NEG // L, unroll=2)
        def grp_body(g):
            n0 = g * L
            s = []
            for j in range(L):
                n = n0 + j
                s.append(tail_v[n, pl.ds(0 * L, L)] * hr0
                         + tail_v[n, pl.ds(1 * L, L)] * hr1
                         + tail_v[n, pl.ds(2 * L, L)] * hr2
                         + tail_v[n, pl.ds(3 * L, L)] * hr3)
            t = [merge(s[2 * i], s[2 * i + 1], 1) for i in range(8)]
            u = [merge(t[2 * i], t[2 * i + 1], 2) for i in range(4)]
            v = [merge(u[2 * i], u[2 * i + 1], 4) for i in range(2)]
            score_v[b, pl.ds(n0, L)] = merge(v[0], v[1], 8)

    start_tails(0, tail0_v, sem0)

    def pair_body(i, carry):
        b0 = 2 * i
        b1 = b0 + 1
        start_tails(b1, tail1_v, sem1)
        wait_tails(b0, tail0_v, sem0)
        compute_row(b0, tail0_v)

        @pl.when(i < BPW // 2 - 1)
        def _():
            start_tails(b0 + 2, tail0_v, sem0)

        wait_tails(b1, tail1_v, sem1)
        compute_row(b1, tail1_v)
        return carry

    lax.fori_loop(0, BPW // 2, pair_body, 0)
    pltpu.sync_copy(score_v, out_hbm.at[pl.ds(base, BPW)])


CB = 256
NBLK_T = NENTITY // CB
NPAIR_T = 61
NEXTRA_T = NBLK_T - 2 * NPAIR_T * NW
NREM = NENTITY - NBLK_T * CB


@functools.partial(
    pl.kernel,
    mesh=_mesh,
    compiler_params=pltpu.CompilerParams(needs_layout_passes=False),
    out_type=jax.ShapeDtypeStruct((NENTITY // 2, 2 * DIM), jnp.float32),
    scratch_types=[
        pltpu.VMEM((DIM, CB), jnp.float32),
        pltpu.VMEM((DIM, CB), jnp.float32),
        pltpu.VMEM((DIM, NREM), jnp.float32),
        pltpu.VMEM((CB // 2, 2 * DIM), jnp.float32),
        pltpu.VMEM((CB // 2, 2 * DIM), jnp.float32),
        pltpu.SemaphoreType.DMA,
        pltpu.SemaphoreType.DMA,
        pltpu.SemaphoreType.DMA,
        pltpu.SemaphoreType.DMA,
    ],
)
def _relayout(entt_hbm, out_hbm, in0, in1, inr, out0, out1,
              rd0, rd1, wr0, wr1):
    wid = lax.axis_index("s") * NC + lax.axis_index("c")
    lanes = lax.iota(jnp.int32, L)
    cc = [g * L + lanes for g in range(CB // L)]
    crow = [lax.shift_right_logical(c, 1) for c in cc]
    ccol = [lax.bitwise_and(c, 1) * DIM for c in cc]

    def read_cp(blk, buf, sem):
        return pltpu.make_async_copy(
            entt_hbm.at[:, pl.ds(blk * CB, CB)], buf, sem)

    def write_cp(blk, buf, sem):
        return pltpu.make_async_copy(
            buf, out_hbm.at[pl.ds(blk * (CB // 2), CB // 2)], sem)

    def transpose_block(inb, outb, ngrp):
        @plsc.parallel_loop(0, DIM, unroll=4)
        def kbody(k):
            dv = lax.bitwise_and(lanes + k, DIM - 1)
            for g in range(ngrp):
                v = plsc.load_gather(inb, [dv, cc[g]])
                plsc.store_scatter(outb, [crow[g], ccol[g] + dv], v)

    read_cp(wid, in0, rd0).start()
    read_cp(wid + NW, in1, rd1).start()

    def pair_body(i, carry):
        j0 = wid + (2 * i) * NW
        j1 = j0 + NW
        read_cp(j0, in0, rd0).wait()

        @pl.when(i > 0)
        def _():
            write_cp(j0 - 2 * NW, out0, wr0).wait()

        transpose_block(in0, out0, CB // L)

        @pl.when(i < NPAIR_T - 1)
        def _():
            read_cp(j0 + 2 * NW, in0, rd0).start()

        write_cp(j0, out0, wr0).start()
        read_cp(j1, in1, rd1).wait()

        @pl.when(i > 0)
        def _():
            write_cp(j1 - 2 * NW, out1, wr1).wait()

        transpose_block(in1, out1, CB // L)

        @pl.when(i < NPAIR_T - 1)
        def _():
            read_cp(j1 + 2 * NW, in1, rd1).start()

        write_cp(j1, out1, wr1).start()
        return carry

    lax.fori_loop(0, NPAIR_T, pair_body, 0)
    last0 = wid + (2 * NPAIR_T - 2) * NW
    write_cp(last0, out0, wr0).wait()
    write_cp(last0 + NW, out1, wr1).wait()

    @pl.when(wid < NEXTRA_T)
    def _():
        jx = 2 * NPAIR_T * NW + wid
        read_cp(jx, in0, rd0).start()
        read_cp(jx, in0, rd0).wait()
        transpose_block(in0, out0, CB // L)
        write_cp(jx, out0, wr0).start()
        write_cp(jx, out0, wr0).wait()

    @pl.when(wid == NW - 1)
    def _():
        c0 = NBLK_T * CB
        rcp = pltpu.make_async_copy(
            entt_hbm.at[:, pl.ds(c0, NREM)], inr, rd1)
        rcp.start()
        rcp.wait()
        transpose_block(inr, out1, NREM // L)
        wcp = pltpu.make_async_copy(
            out1.at[pl.ds(0, NREM // 2)],
            out_hbm.at[pl.ds(c0 // 2, NREM // 2)], wr1)
        wcp.start()
        wcp.wait()


def kernel(pos_part, neg_part, entity_embedding, relation_embedding):
    hidx = pos_part[:, 0].astype(jnp.int32)
    ridx = pos_part[:, 1].astype(jnp.int32)
    neg3 = neg_part.astype(jnp.int32).reshape(BATCH, 2, NEG_HALF)
    ent_lin = _relayout(entity_embedding.T).reshape(NENTITY, DIM)
    return _kge_sc(hidx, ridx, neg3, ent_lin, relation_embedding)
